# baseline (speedup 1.0000x reference)
import functools

import jax
import jax.numpy as jnp
from jax import lax
from jax.experimental import pallas as pl
from jax.experimental.pallas import tpu as pltpu
from jax.experimental.pallas import tpu_sc as plsc

T, D, E, K, H = 2048, 1024, 8, 2, 4096
BM = 256
HTILE = 512
HT = H // HTILE
P = K * T + E * BM
NB = P // BM
NC, NS = 2, 16
NW = NC * NS
ROWS_G = P // NW
CH = 64
TW = T // NW
CT = 16


def _router(xf, Wg):
    gate_logits = xf @ Wg.T
    gate_probs = jax.nn.softmax(gate_logits, axis=-1)
    rw, ei = lax.top_k(gate_probs, K)
    rw = rw / rw.sum(axis=-1, keepdims=True)
    return gate_probs, rw, ei


def _dispatch(rw, ei):
    ee = ei.reshape(-1)
    onehot = (ee[:, None] == jnp.arange(E, dtype=ee.dtype)[None, :])
    oh = onehot.astype(jnp.int32)
    incl = jnp.cumsum(oh, axis=0)
    rank = jnp.sum(incl * oh, axis=1) - 1
    counts = incl[-1]
    padded = ((counts + BM - 1) // BM) * BM
    ends = jnp.cumsum(padded)
    starts = ends - padded
    starts_ee = jnp.sum(oh * starts[None, :], axis=1)
    posf = (starts_ee + rank).astype(jnp.int32)
    total = ends[-1]
    bstart = jnp.arange(NB, dtype=jnp.int32) * BM
    act = (bstart < total).astype(jnp.int32)
    ebf = jnp.sum(bstart[:, None] >= ends[None, :], axis=1).astype(jnp.int32)
    eb_last = jnp.max(jnp.where(act == 1, ebf, -1))
    eb = jnp.where(act == 1, ebf, eb_last).astype(jnp.int32)
    prev_eb = jnp.concatenate([eb[:1] - 1, eb[:-1]])
    chg = (act * (eb != prev_eb)).astype(jnp.int32)
    rid = jnp.cumsum(chg) - 1
    par = (rid % 2).astype(jnp.int32)
    bidx = jnp.arange(NB, dtype=jnp.int32)
    posns = jnp.where(chg == 1, bidx, 2 * NB)
    sufmin = lax.associative_scan(jnp.minimum, posns[::-1])[::-1]
    nxt_pos = jnp.concatenate([sufmin[1:], jnp.full((1,), 2 * NB, jnp.int32)])
    hasnxt = (chg * (nxt_pos < NB)).astype(jnp.int32)
    nxt = jnp.sum((nxt_pos[:, None] == bidx[None, :]) * eb[None, :],
                  axis=1).astype(jnp.int32)
    pos0 = posf[0::2]
    pos1 = posf[1::2]
    return eb, act, chg, par, hasnxt, nxt, pos0, pos1, counts


def _mlp_body(eb_ref, act_ref, chg_ref, par_ref, hasnxt_ref, nxt_ref,
              x_ref, wfc_hbm, wpr_hbm, o_ref, h_ref, wfc_v, wp_v, sfc, spr):
    b = pl.program_id(0)

    QF = 4
    FCH = H // QF

    def wfc_chunk(e, slot, q):
        return pltpu.make_async_copy(
            wfc_hbm.at[e].at[pl.ds(q * FCH, FCH), :],
            wfc_v.at[slot].at[pl.ds(q * FCH, FCH), :], sfc)

    def wpr_chunk(e, t):
        return pltpu.make_async_copy(
            wpr_hbm.at[e].at[:, pl.ds(t * HTILE, HTILE)],
            wp_v.at[:, pl.ds(t * HTILE, HTILE)], spr.at[t])

    @pl.when(b == 0)
    def _():
        for q in range(QF):
            wfc_chunk(eb_ref[0], 0, q).start()

    @pl.when(chg_ref[b] == 1)
    def _():
        for q in range(QF):
            wfc_chunk(eb_ref[b], par_ref[b], q).wait()
        for t in range(HT):
            wpr_chunk(eb_ref[b], t).start()

    @pl.when(hasnxt_ref[b] == 1)
    def _():
        for q in range(QF):
            wfc_chunk(nxt_ref[b], 1 - par_ref[b], q).start()

    @pl.when(act_ref[b] == 1)
    def _():
        x = x_ref[...]
        p = par_ref[b]
        for t in range(HT):
            h = lax.dot_general(x, wfc_v[p, pl.ds(t * HTILE, HTILE), :],
                                (((1,), (1,)), ((), ())),
                                preferred_element_type=jnp.float32)
            h_ref[:, pl.ds(t * HTILE, HTILE)] = jnp.square(jnp.maximum(h, 0.0))

        @pl.when(chg_ref[b] == 1)
        def _():
            for t in range(HT):
                wpr_chunk(eb_ref[b], t).wait()

        o_ref[...] = lax.dot_general(h_ref[...], wp_v[...],
                                     (((1,), (1,)), ((), ())),
                                     preferred_element_type=jnp.float32)


def _mlp_call(eb, act, chg, par, hasnxt, nxt, xs, Wfc, Wproj, interpret=False):
    idx = lambda b, *refs: (b, 0)
    return pl.pallas_call(
        _mlp_body,
        grid_spec=pltpu.PrefetchScalarGridSpec(
            num_scalar_prefetch=6,
            grid=(NB,),
            in_specs=[
                pl.BlockSpec((BM, D), idx),
                pl.BlockSpec(memory_space=pl.ANY),
                pl.BlockSpec(memory_space=pl.ANY),
            ],
            out_specs=pl.BlockSpec((BM, D), idx),
            scratch_shapes=[
                pltpu.VMEM((BM, H), jnp.float32),
                pltpu.VMEM((2, H, D), jnp.float32),
                pltpu.VMEM((D, H), jnp.float32),
                pltpu.SemaphoreType.DMA,
                pltpu.SemaphoreType.DMA((HT,)),
            ],
        ),
        out_shape=jax.ShapeDtypeStruct((P, D), jnp.float32),
        compiler_params=pltpu.CompilerParams(
            vmem_limit_bytes=62 * 1024 * 1024),
        interpret=interpret,
    )(eb, act, chg, par, hasnxt, nxt, xs, Wfc, Wproj)


def _sc_dispatch_call(pos0, pos1, xf):
    mesh = plsc.VectorSubcoreMesh(core_axis_name="c", subcore_axis_name="s")

    @functools.partial(
        pl.kernel,
        mesh=mesh,
        out_type=jax.ShapeDtypeStruct((P, D), jnp.float32),
        scratch_types=[
            pltpu.VMEM((TW,), jnp.int32),
            pltpu.VMEM((TW,), jnp.int32),
            pltpu.VMEM((TW, D), jnp.float32),
            pltpu.SemaphoreType.DMA,
            pltpu.SemaphoreType.DMA,
        ],
    )
    def k(p0_hbm, p1_hbm, x_hbm, out_hbm, p0_v, p1_v, rows_v, s0, s1):
        wid = lax.axis_index("s") * NC + lax.axis_index("c")
        tb = wid * TW
        pltpu.sync_copy(p0_hbm.at[pl.ds(tb, TW)], p0_v)
        pltpu.sync_copy(p1_hbm.at[pl.ds(tb, TW)], p1_v)
        pltpu.sync_copy(x_hbm.at[pl.ds(tb, TW)], rows_v)
        c0 = pltpu.async_copy(rows_v, out_hbm.at[p0_v], s0)
        c1 = pltpu.async_copy(rows_v, out_hbm.at[p1_v], s1)
        c0.wait()
        c1.wait()

    return k(pos0, pos1, xf)


def _sc_combine_call(pos0, pos1, w0, w1, ys):
    mesh = plsc.VectorSubcoreMesh(core_axis_name="c", subcore_axis_name="s")

    @functools.partial(
        pl.kernel,
        mesh=mesh,
        out_type=jax.ShapeDtypeStruct((T, D), jnp.float32),
        scratch_types=[
            pltpu.VMEM((TW,), jnp.int32),
            pltpu.VMEM((TW,), jnp.int32),
            pltpu.VMEM((TW, 16), jnp.float32),
            pltpu.VMEM((TW, 16), jnp.float32),
            pltpu.VMEM((2, CT, D), jnp.float32),
            pltpu.VMEM((2, CT, D), jnp.float32),
            pltpu.SemaphoreType.DMA((2,)),
            pltpu.SemaphoreType.DMA((2,)),
        ],
    )
    def k(p0_hbm, p1_hbm, w0_hbm, w1_hbm, ys_hbm, out_hbm,
          p0_v, p1_v, w0_v, w1_v, a_v, b_v, sg, so):
        wid = lax.axis_index("s") * NC + lax.axis_index("c")
        base = wid * TW
        nch = TW // CT
        pltpu.sync_copy(p0_hbm.at[pl.ds(base, TW)], p0_v)
        pltpu.sync_copy(p1_hbm.at[pl.ds(base, TW)], p1_v)
        pltpu.sync_copy(w0_hbm.at[pl.ds(base, TW)], w0_v)
        pltpu.sync_copy(w1_hbm.at[pl.ds(base, TW)], w1_v)

        def gathers(c):
            ping = c % 2
            ca = pltpu.make_async_copy(
                ys_hbm.at[p0_v.at[pl.ds(c * CT, CT)]], a_v.at[ping],
                sg.at[ping])
            cb = pltpu.make_async_copy(
                ys_hbm.at[p1_v.at[pl.ds(c * CT, CT)]], b_v.at[ping],
                sg.at[ping])
            return ca, cb

        def out_copy(c):
            ping = c % 2
            return pltpu.make_async_copy(
                a_v.at[ping], out_hbm.at[pl.ds(base + c * CT, CT)],
                so.at[ping])

        ca, cb = gathers(0)
        ca.start()
        cb.start()
        for c in range(nch):
            ping = c % 2
            if c + 1 < nch:
                if c >= 1:
                    out_copy(c - 1).wait()
                ca, cb = gathers(c + 1)
                ca.start()
                cb.start()
            ca0, cb0 = gathers(c)
            ca0.wait()
            cb0.wait()

            def row(r, carry):
                wa = w0_v[c * CT + r]
                wb = w1_v[c * CT + r]

                def col(j, carry2):
                    sl = pl.ds(j * 16, 16)
                    a_v[ping, r, sl] = (wa * a_v[ping, r, sl]
                                        + wb * b_v[ping, r, sl])
                    return carry2
                return lax.fori_loop(0, D // 16, col, carry)

            lax.fori_loop(0, CT, row, 0)
            out_copy(c).start()
        out_copy(nch - 2).wait()
        out_copy(nch - 1).wait()

    return k(pos0, pos1, w0, w1, ys)


def kernel(x, Wg, Wfc, Wproj):
    Bv, Tv, Dv = x.shape
    xf = x.reshape(Tv * Bv, Dv)
    gate_probs, rw, ei = _router(xf, Wg)
    eb, act, chg, par, hasnxt, nxt, pos0, pos1, counts = _dispatch(rw, ei)
    xs = _sc_dispatch_call(pos0, pos1, xf)
    ys = _mlp_call(eb, act, chg, par, hasnxt, nxt, xs, Wfc, Wproj)
    w0b = jnp.broadcast_to(rw[:, :1], (T, 16))
    w1b = jnp.broadcast_to(rw[:, 1:], (T, 16))
    out = _sc_combine_call(pos0, pos1, w0b, w1b, ys)
    freq = counts.astype(jnp.float32) / T
    balance_loss = (gate_probs.mean(axis=0) * freq).sum() * E
    return out.reshape(Bv, Tv, Dv), balance_loss


# speedup vs baseline: 1.1169x; 1.1169x over previous
import functools

import jax
import jax.numpy as jnp
from jax import lax
from jax.experimental import pallas as pl
from jax.experimental.pallas import tpu as pltpu
from jax.experimental.pallas import tpu_sc as plsc

T, D, E, K, H = 2048, 1024, 8, 2, 4096
BM = 256
HTILE = 512
HT = H // HTILE
P = K * T + E * BM
NB = P // BM
NC, NS = 2, 16
NW = NC * NS
ROWS_G = P // NW
CH = 64
TW = T // NW
CT = 16


def _router(xf, Wg):
    gate_logits = xf @ Wg.T
    gate_probs = jax.nn.softmax(gate_logits, axis=-1)
    rw, ei = lax.top_k(gate_probs, K)
    rw = rw / rw.sum(axis=-1, keepdims=True)
    return gate_probs, rw, ei


def _dispatch(rw, ei):
    ee = ei.reshape(-1)
    onehot = (ee[:, None] == jnp.arange(E, dtype=ee.dtype)[None, :])
    oh = onehot.astype(jnp.int32)
    incl = jnp.cumsum(oh, axis=0)
    rank = jnp.sum(incl * oh, axis=1) - 1
    counts = incl[-1]
    padded = ((counts + BM - 1) // BM) * BM
    ends = jnp.cumsum(padded)
    starts = ends - padded
    starts_ee = jnp.sum(oh * starts[None, :], axis=1)
    posf = (starts_ee + rank).astype(jnp.int32)
    total = ends[-1]
    bstart = jnp.arange(NB, dtype=jnp.int32) * BM
    act = (bstart < total).astype(jnp.int32)
    ebf = jnp.sum(bstart[:, None] >= ends[None, :], axis=1).astype(jnp.int32)
    eb_last = jnp.max(jnp.where(act == 1, ebf, -1))
    eb = jnp.where(act == 1, ebf, eb_last).astype(jnp.int32)
    prev_eb = jnp.concatenate([eb[:1] - 1, eb[:-1]])
    chg = (act * (eb != prev_eb)).astype(jnp.int32)
    rid = jnp.cumsum(chg) - 1
    par = (rid % 2).astype(jnp.int32)
    bidx = jnp.arange(NB, dtype=jnp.int32)
    posns = jnp.where(chg == 1, bidx, 2 * NB)
    sufmin = lax.associative_scan(jnp.minimum, posns[::-1])[::-1]
    nxt_pos = jnp.concatenate([sufmin[1:], jnp.full((1,), 2 * NB, jnp.int32)])
    hasnxt = (chg * (nxt_pos < NB)).astype(jnp.int32)
    nxt = jnp.sum((nxt_pos[:, None] == bidx[None, :]) * eb[None, :],
                  axis=1).astype(jnp.int32)
    pos0 = posf[0::2]
    pos1 = posf[1::2]
    return eb, act, chg, par, hasnxt, nxt, pos0, pos1, counts


def _mlp_body(eb_ref, act_ref, chg_ref, par_ref, hasnxt_ref, nxt_ref,
              x_ref, wfc_hbm, wpr_hbm, o_ref, h_ref, wfc_v, wp_v, sfc, spr):
    b = pl.program_id(0)

    @pl.when(b == 0)
    def _():
        pltpu.make_async_copy(wfc_hbm.at[eb_ref[0]], wfc_v.at[0], sfc).start()

    @pl.when(chg_ref[b] == 1)
    def _():
        pltpu.make_async_copy(wfc_hbm.at[eb_ref[b]],
                              wfc_v.at[par_ref[b]], sfc).wait()
        pltpu.make_async_copy(wpr_hbm.at[eb_ref[b]], wp_v, spr).start()

    @pl.when(hasnxt_ref[b] == 1)
    def _():
        pltpu.make_async_copy(wfc_hbm.at[nxt_ref[b]],
                              wfc_v.at[1 - par_ref[b]], sfc).start()

    @pl.when(act_ref[b] == 1)
    def _():
        x = x_ref[...]
        p = par_ref[b]
        for t in range(HT):
            h = lax.dot_general(x, wfc_v[p, pl.ds(t * HTILE, HTILE), :],
                                (((1,), (1,)), ((), ())),
                                preferred_element_type=jnp.float32)
            h_ref[:, pl.ds(t * HTILE, HTILE)] = jnp.square(jnp.maximum(h, 0.0))

        @pl.when(chg_ref[b] == 1)
        def _():
            pltpu.make_async_copy(wpr_hbm.at[eb_ref[b]], wp_v, spr).wait()

        o_ref[...] = lax.dot_general(h_ref[...], wp_v[...],
                                     (((1,), (1,)), ((), ())),
                                     preferred_element_type=jnp.float32)


def _mlp_call(eb, act, chg, par, hasnxt, nxt, xs, Wfc, Wproj, interpret=False):
    idx = lambda b, *refs: (b, 0)
    return pl.pallas_call(
        _mlp_body,
        grid_spec=pltpu.PrefetchScalarGridSpec(
            num_scalar_prefetch=6,
            grid=(NB,),
            in_specs=[
                pl.BlockSpec((BM, D), idx),
                pl.BlockSpec(memory_space=pl.ANY),
                pl.BlockSpec(memory_space=pl.ANY),
            ],
            out_specs=pl.BlockSpec((BM, D), idx),
            scratch_shapes=[
                pltpu.VMEM((BM, H), jnp.float32),
                pltpu.VMEM((2, H, D), jnp.float32),
                pltpu.VMEM((D, H), jnp.float32),
                pltpu.SemaphoreType.DMA,
                pltpu.SemaphoreType.DMA,
            ],
        ),
        out_shape=jax.ShapeDtypeStruct((P, D), jnp.float32),
        compiler_params=pltpu.CompilerParams(
            vmem_limit_bytes=62 * 1024 * 1024),
        interpret=interpret,
    )(eb, act, chg, par, hasnxt, nxt, xs, Wfc, Wproj)


def _sc_dispatch_call(pos0, pos1, xf):
    mesh = plsc.VectorSubcoreMesh(core_axis_name="c", subcore_axis_name="s")

    @functools.partial(
        pl.kernel,
        mesh=mesh,
        out_type=jax.ShapeDtypeStruct((P, D), jnp.float32),
        scratch_types=[
            pltpu.VMEM((TW,), jnp.int32),
            pltpu.VMEM((TW,), jnp.int32),
            pltpu.VMEM((TW, D), jnp.float32),
            pltpu.SemaphoreType.DMA,
            pltpu.SemaphoreType.DMA,
        ],
    )
    def k(p0_hbm, p1_hbm, x_hbm, out_hbm, p0_v, p1_v, rows_v, s0, s1):
        wid = lax.axis_index("s") * NC + lax.axis_index("c")
        tb = wid * TW
        pltpu.sync_copy(p0_hbm.at[pl.ds(tb, TW)], p0_v)
        pltpu.sync_copy(p1_hbm.at[pl.ds(tb, TW)], p1_v)
        pltpu.sync_copy(x_hbm.at[pl.ds(tb, TW)], rows_v)
        c0 = pltpu.async_copy(rows_v, out_hbm.at[p0_v], s0)
        c1 = pltpu.async_copy(rows_v, out_hbm.at[p1_v], s1)
        c0.wait()
        c1.wait()

    return k(pos0, pos1, xf)


def _sc_combine_call(pos0, pos1, w0, w1, ys):
    mesh = plsc.VectorSubcoreMesh(core_axis_name="c", subcore_axis_name="s")

    @functools.partial(
        pl.kernel,
        mesh=mesh,
        out_type=jax.ShapeDtypeStruct((T, D), jnp.float32),
        scratch_types=[
            pltpu.VMEM((TW,), jnp.int32),
            pltpu.VMEM((TW,), jnp.int32),
            pltpu.VMEM((TW, 16), jnp.float32),
            pltpu.VMEM((TW, 16), jnp.float32),
            pltpu.VMEM((2, CT, D), jnp.float32),
            pltpu.VMEM((2, CT, D), jnp.float32),
            pltpu.SemaphoreType.DMA((2,)),
            pltpu.SemaphoreType.DMA((2,)),
        ],
    )
    def k(p0_hbm, p1_hbm, w0_hbm, w1_hbm, ys_hbm, out_hbm,
          p0_v, p1_v, w0_v, w1_v, a_v, b_v, sg, so):
        wid = lax.axis_index("s") * NC + lax.axis_index("c")
        base = wid * TW
        nch = TW // CT
        pltpu.sync_copy(p0_hbm.at[pl.ds(base, TW)], p0_v)
        pltpu.sync_copy(p1_hbm.at[pl.ds(base, TW)], p1_v)
        pltpu.sync_copy(w0_hbm.at[pl.ds(base, TW)], w0_v)
        pltpu.sync_copy(w1_hbm.at[pl.ds(base, TW)], w1_v)

        def gathers(c):
            ping = c % 2
            ca = pltpu.make_async_copy(
                ys_hbm.at[p0_v.at[pl.ds(c * CT, CT)]], a_v.at[ping],
                sg.at[ping])
            cb = pltpu.make_async_copy(
                ys_hbm.at[p1_v.at[pl.ds(c * CT, CT)]], b_v.at[ping],
                sg.at[ping])
            return ca, cb

        def out_copy(c):
            ping = c % 2
            return pltpu.make_async_copy(
                a_v.at[ping], out_hbm.at[pl.ds(base + c * CT, CT)],
                so.at[ping])

        ca, cb = gathers(0)
        ca.start()
        cb.start()
        for c in range(nch):
            ping = c % 2
            if c + 1 < nch:
                if c >= 1:
                    out_copy(c - 1).wait()
                ca, cb = gathers(c + 1)
                ca.start()
                cb.start()
            ca0, cb0 = gathers(c)
            ca0.wait()
            cb0.wait()

            def row(r, carry):
                wa = w0_v[c * CT + r]
                wb = w1_v[c * CT + r]

                def col(j, carry2):
                    sl = pl.ds(j * 16, 16)
                    a_v[ping, r, sl] = (wa * a_v[ping, r, sl]
                                        + wb * b_v[ping, r, sl])
                    return carry2
                return lax.fori_loop(0, D // 16, col, carry)

            lax.fori_loop(0, CT, row, 0)
            out_copy(c).start()
        out_copy(nch - 2).wait()
        out_copy(nch - 1).wait()

    return k(pos0, pos1, w0, w1, ys)


def kernel(x, Wg, Wfc, Wproj):
    Bv, Tv, Dv = x.shape
    xf = x.reshape(Tv * Bv, Dv)
    gate_probs, rw, ei = _router(xf, Wg)
    eb, act, chg, par, hasnxt, nxt, pos0, pos1, counts = _dispatch(rw, ei)
    xs = _sc_dispatch_call(pos0, pos1, xf)
    ys = _mlp_call(eb, act, chg, par, hasnxt, nxt, xs, Wfc, Wproj)
    w0b = jnp.broadcast_to(rw[:, :1], (T, 16))
    w1b = jnp.broadcast_to(rw[:, 1:], (T, 16))
    out = _sc_combine_call(pos0, pos1, w0b, w1b, ys)
    freq = counts.astype(jnp.float32) / T
    balance_loss = (gate_probs.mean(axis=0) * freq).sum() * E
    return out.reshape(Bv, Tv, Dv), balance_loss
